# baseline (speedup 1.0000x reference)
import functools

import jax
import jax.numpy as jnp
from jax import lax
from jax.experimental import pallas as pl
from jax.experimental.pallas import tpu as pltpu
from jax.experimental.pallas import tpu_sc as plsc

B = 16384
L = 50
LP = 64
D = 64
NC = 2
NS = 16
NW = NC * NS
PW = B // NW
NLANE = 16
ND = D // NLANE
LG = 56


def _qenc_body(ids_hbm, w_hbm, out_hbm, ids_v, rows0, rows1, out_v, sem0, sem1):
    wid = lax.axis_index("s") * NC + lax.axis_index("c")
    base = wid * PW
    pltpu.sync_copy(ids_hbm.at[pl.ds(base, PW)], ids_v)

    def issue(s, buf, sem):
        pltpu.async_copy(w_hbm.at[ids_v.at[s, pl.ds(0, LG)]], buf, sem)

    def drain(s, buf, sem):
        pltpu.make_async_copy(w_hbm.at[ids_v.at[s, pl.ds(0, LG)]], buf, sem).wait()

    def reduce_one(s, buf):
        accs = [buf[0, pl.ds(d * NLANE, NLANE)] for d in range(ND)]
        for l in range(1, L):
            for d in range(ND):
                accs[d] = accs[d] + buf[l, pl.ds(d * NLANE, NLANE)]
        cnt = plsc.all_reduce_population_count(ids_v[s, pl.ds(0, NLANE)] != 0)
        for q in range(1, LP // NLANE):
            cnt = cnt + plsc.all_reduce_population_count(
                ids_v[s, pl.ds(q * NLANE, NLANE)] != 0)
        safe = jnp.maximum(cnt.astype(jnp.float32), 1.0)
        for d in range(ND):
            out_v[s, pl.ds(d * NLANE, NLANE)] = accs[d] / safe

    issue(0, rows0, sem0)

    def body(g, carry):
        s0 = 2 * g
        s1 = s0 + 1
        issue(s1, rows1, sem1)
        drain(s0, rows0, sem0)
        reduce_one(s0, rows0)

        @pl.when(s1 + 1 < PW)
        def _():
            issue(s1 + 1, rows0, sem0)

        drain(s1, rows1, sem1)
        reduce_one(s1, rows1)
        return carry

    lax.fori_loop(0, PW // 2, body, 0)
    pltpu.sync_copy(out_v, out_hbm.at[pl.ds(base, PW)])


@jax.jit
def _qenc(ids_pad, w):
    mesh = plsc.VectorSubcoreMesh(core_axis_name="c", subcore_axis_name="s")
    f = functools.partial(
        pl.kernel,
        mesh=mesh,
        compiler_params=pltpu.CompilerParams(
            needs_layout_passes=False, use_tc_tiling_on_sc=False),
        out_type=jax.ShapeDtypeStruct((B, D), jnp.float32),
        scratch_types=[
            pltpu.VMEM((PW, LP), jnp.int32),
            pltpu.VMEM((LG, D), jnp.float32),
            pltpu.VMEM((LG, D), jnp.float32),
            pltpu.VMEM((PW, D), jnp.float32),
            pltpu.SemaphoreType.DMA,
            pltpu.SemaphoreType.DMA,
        ],
    )(_qenc_body)
    return f(ids_pad, w)


def kernel(seqs, W):
    ids_pad = jnp.pad(seqs, ((0, 0), (0, LP - L)))
    return _qenc(ids_pad, W)


# speedup vs baseline: 3.5326x; 3.5326x over previous
import functools

import jax
import jax.numpy as jnp
from jax import lax
from jax.experimental import pallas as pl
from jax.experimental.pallas import tpu as pltpu
from jax.experimental.pallas import tpu_sc as plsc

B = 16384
L = 50
D = 64
NC = 2
NS = 16
NW = NC * NS
PW = B // NW
NID = PW * L
NLANE = 16
ND = D // NLANE
SPP = 16
RING = SPP * L
NP = PW // SPP
HALF = RING // 2
GROUP = 200
SPLITS = ((0, 104), (104, 96))


def _qenc_body(ids_hbm, w_hbm, out_hbm, idsf_v, ring_v, out_v, semA, semB):
    wid = lax.axis_index("s") * NC + lax.axis_index("c")
    sbase = wid * PW
    pltpu.sync_copy(ids_hbm.at[pl.ds(wid * NID, NID)], idsf_v)

    def half_streams(p, half):
        base = pl.multiple_of(RING * p + HALF * half, 8)
        out = []
        for g in range(HALF // GROUP):
            for off, size in SPLITS:
                src = w_hbm.at[idsf_v.at[pl.ds(base + GROUP * g + off, size)]]
                dst = ring_v.at[pl.ds(HALF * half + GROUP * g + off, size)]
                out.append((src, dst))
        return out

    def issue(p, half, sem):
        for src, dst in half_streams(p, half):
            pltpu.async_copy(src, dst, sem)

    def drain(p, half, sem):
        for src, dst in half_streams(p, half):
            pltpu.make_async_copy(src, dst, sem).wait()

    lane = lax.iota(jnp.int32, NLANE)

    def reduce_half(p, half):
        def one(j, carry):
            rb = HALF * half + L * j
            accs = [ring_v[rb, pl.ds(d * NLANE, NLANE)] for d in range(ND)]
            for l in range(1, L):
                for d in range(ND):
                    accs[d] = accs[d] + ring_v[rb + l, pl.ds(d * NLANE, NLANE)]
            s = SPP * p + 8 * half + j
            ib = L * s
            cnt = plsc.all_reduce_population_count(
                plsc.load_gather(idsf_v, [ib + lane]) != 0)
            for q in range(1, 3):
                cnt = cnt + plsc.all_reduce_population_count(
                    plsc.load_gather(idsf_v, [ib + q * NLANE + lane]) != 0)
            tailmask = lane < (L - 3 * NLANE)
            tail = plsc.load_gather(
                idsf_v, [ib + jnp.where(tailmask, 3 * NLANE + lane, 0)])
            cnt = cnt + plsc.all_reduce_population_count((tail != 0) & tailmask)
            safe = jnp.maximum(cnt.astype(jnp.float32), 1.0)
            for d in range(ND):
                out_v[s, pl.ds(d * NLANE, NLANE)] = accs[d] / safe
            return carry

        lax.fori_loop(0, SPP // 2, one, 0)

    issue(0, 0, semA)
    issue(0, 1, semB)

    def body(p, carry):
        drain(p, 0, semA)
        reduce_half(p, 0)

        @pl.when(p + 1 < NP)
        def _():
            issue(p + 1, 0, semA)

        drain(p, 1, semB)
        reduce_half(p, 1)

        @pl.when(p + 1 < NP)
        def _():
            issue(p + 1, 1, semB)

        return carry

    lax.fori_loop(0, NP, body, 0)
    pltpu.sync_copy(out_v, out_hbm.at[pl.ds(sbase, PW)])


@jax.jit
def _qenc(ids_flat, w):
    mesh = plsc.VectorSubcoreMesh(core_axis_name="c", subcore_axis_name="s")
    f = functools.partial(
        pl.kernel,
        mesh=mesh,
        compiler_params=pltpu.CompilerParams(
            needs_layout_passes=False, use_tc_tiling_on_sc=False),
        out_type=jax.ShapeDtypeStruct((B, D), jnp.float32),
        scratch_types=[
            pltpu.VMEM((NID,), jnp.int32),
            pltpu.VMEM((RING, D), jnp.float32),
            pltpu.VMEM((PW, D), jnp.float32),
            pltpu.SemaphoreType.DMA,
            pltpu.SemaphoreType.DMA,
        ],
    )(_qenc_body)
    return f(ids_flat, w)


def kernel(seqs, W):
    return _qenc(seqs.reshape(-1), W)
